# baseline (speedup 1.0000x reference)
import jax
import jax.numpy as jnp
from jax.experimental import pallas as pl
from jax.experimental.pallas import tpu as pltpu

_PACK = 4
_TB = 4096


def _mlp_chain(xp, w1_ref, w2_ref, w3_ref, b1_ref, b2_ref, b3_ref):
    h = jnp.dot(xp, w1_ref[...], preferred_element_type=jnp.float32)
    h = jnp.maximum(h + b1_ref[...], 0.0).astype(jnp.bfloat16)
    h = jnp.dot(h, w2_ref[...], preferred_element_type=jnp.float32)
    h = jnp.maximum(h + b2_ref[...], 0.0).astype(jnp.bfloat16)
    h = jnp.dot(h, w3_ref[...], preferred_element_type=jnp.float32)
    return jnp.maximum(h + b3_ref[...], 0.0).astype(jnp.bfloat16)


def _raw_body(x0_ref, x1_ref, x2_ref, x3_ref,
              w1_ref, w2_ref, w3_ref, w4_ref,
              b1_ref, b2_ref, b3_ref, b4_ref, out_ref):
    tb = x0_ref.shape[0]
    xp = jnp.concatenate(
        [x0_ref[...], x1_ref[...], x2_ref[...], x3_ref[...]], axis=1)
    h = _mlp_chain(xp, w1_ref, w2_ref, w3_ref, b1_ref, b2_ref, b3_ref)
    for j in range(_PACK):
        q = jnp.dot(h[:, 32 * j:32 * j + 32], w4_ref[...],
                    preferred_element_type=jnp.float32)
        out_ref[pl.ds(j * tb, tb), :] = q + b4_ref[...]


def _packed_body(xr_ref, w1_ref, w2_ref, w3_ref, w4_ref,
                 b1_ref, b2_ref, b3_ref, b4_ref, _alias_ref, out_ref):
    tb = xr_ref.shape[0]
    h = _mlp_chain(xr_ref[...], w1_ref, w2_ref, w3_ref,
                   b1_ref, b2_ref, b3_ref)
    for j in range(_PACK):
        q = jnp.dot(h[:, 32 * j:32 * j + 32], w4_ref[...],
                    preferred_element_type=jnp.float32)
        out_ref[slice(j, None, _PACK), :] = q + b4_ref[...]


def _pack_block_diag(w1, w2, w3, b1, b2, b3):
    w1bd = jnp.zeros((16, 64), jnp.float32)
    w2bd = jnp.zeros((64, 128), jnp.float32)
    w3bd = jnp.zeros((128, 128), jnp.float32)
    for i in range(_PACK):
        w1bd = w1bd.at[4 * i:4 * i + 4, 16 * i:16 * i + 16].set(w1)
        w2bd = w2bd.at[16 * i:16 * i + 16, 32 * i:32 * i + 32].set(w2)
        w3bd = w3bd.at[32 * i:32 * i + 32, 32 * i:32 * i + 32].set(w3)
    b1t = jnp.tile(b1, _PACK)[None, :]
    b2t = jnp.tile(b2, _PACK)[None, :]
    b3t = jnp.tile(b3, _PACK)[None, :]
    return (w1bd, w2bd.astype(jnp.bfloat16), w3bd.astype(jnp.bfloat16),
            b1t, b2t, b3t)


@jax.jit
def _dqn_fused(x, w1, b1, w2, b2, w3, b3, w4, b4):
    batch = x.shape[0]
    rows = _PACK * _TB
    half = (batch // 2) // rows * rows

    w1bd, w2bd, w3bd, b1t, b2t, b3t = _pack_block_diag(w1, w2, w3, b1, b2, b3)
    w4b = w4.astype(jnp.bfloat16)
    b4t = b4[None, :]
    weights = (w1bd, w2bd, w3bd, w4b, b1t, b2t, b3t, b4t)
    resident = lambda shape: pl.BlockSpec(shape, lambda i: (0,) * len(shape))
    weight_specs = [resident(w.shape) for w in weights]

    xr = x[half:].reshape((batch - half) // _PACK, _PACK * x.shape[1])

    def chunk_spec(j):
        return pl.BlockSpec((_TB, x.shape[1]),
                            lambda i, j=j: (_PACK * i + j, 0))

    out_a = pl.pallas_call(
        _raw_body,
        out_shape=jax.ShapeDtypeStruct((batch, 2), jnp.float32),
        grid=(half // rows,),
        in_specs=[chunk_spec(0), chunk_spec(1), chunk_spec(2), chunk_spec(3),
                  *weight_specs],
        out_specs=pl.BlockSpec((rows, 2), lambda i: (i, 0)),
        compiler_params=pltpu.CompilerParams(
            dimension_semantics=("parallel",)),
    )(x, x, x, x, *weights)

    n_b = (batch - half) // rows
    return pl.pallas_call(
        _packed_body,
        out_shape=jax.ShapeDtypeStruct((batch, 2), jnp.float32),
        grid=(n_b,),
        in_specs=[pl.BlockSpec((_TB, _PACK * x.shape[1]), lambda i: (i, 0)),
                  *weight_specs,
                  pl.BlockSpec((8, 2), lambda i: (0, 0))],
        out_specs=pl.BlockSpec((rows, 2),
                               lambda i: (half // rows + i, 0)),
        input_output_aliases={9: 0},
        compiler_params=pltpu.CompilerParams(
            dimension_semantics=("parallel",)),
    )(xr, *weights, out_a)


def kernel(x, w1, b1, w2, b2, w3, b3, w4, b4):
    return _dqn_fused(x, w1, b1, w2, b2, w3, b3, w4, b4)


# speedup vs baseline: 1.5211x; 1.5211x over previous
import jax
import jax.numpy as jnp
from jax.experimental import pallas as pl
from jax.experimental.pallas import tpu as pltpu

_PACK = 4
_TB = 4096


def _mlp_body(x0_ref, x1_ref, x2_ref, x3_ref,
              w1_ref, w2_ref, w3_ref, w4_ref,
              b1_ref, b2_ref, b3_ref, b4_ref, out_ref):
    tb = x0_ref.shape[0]
    xp = jnp.concatenate(
        [x0_ref[...], x1_ref[...], x2_ref[...], x3_ref[...]], axis=1)

    h = jnp.dot(xp, w1_ref[...], preferred_element_type=jnp.float32)
    h = jnp.maximum(h + b1_ref[...], 0.0).astype(jnp.bfloat16)
    h = jnp.dot(h, w2_ref[...], preferred_element_type=jnp.float32)
    h = jnp.maximum(h + b2_ref[...], 0.0).astype(jnp.bfloat16)
    h = jnp.dot(h, w3_ref[...], preferred_element_type=jnp.float32)
    h = jnp.maximum(h + b3_ref[...], 0.0).astype(jnp.bfloat16)

    for j in range(_PACK):
        q = jnp.dot(h[:, 32 * j:32 * j + 32], w4_ref[...],
                    preferred_element_type=jnp.float32)
        out_ref[pl.ds(j * tb, tb), :] = q + b4_ref[...]


def _pack_block_diag(w1, w2, w3, b1, b2, b3):
    w1bd = jnp.zeros((16, 64), jnp.float32)
    w2bd = jnp.zeros((64, 128), jnp.float32)
    w3bd = jnp.zeros((128, 128), jnp.float32)
    for i in range(_PACK):
        w1bd = w1bd.at[4 * i:4 * i + 4, 16 * i:16 * i + 16].set(w1)
        w2bd = w2bd.at[16 * i:16 * i + 16, 32 * i:32 * i + 32].set(w2)
        w3bd = w3bd.at[32 * i:32 * i + 32, 32 * i:32 * i + 32].set(w3)
    b1t = jnp.tile(b1, _PACK)[None, :]
    b2t = jnp.tile(b2, _PACK)[None, :]
    b3t = jnp.tile(b3, _PACK)[None, :]
    return (w1bd, w2bd.astype(jnp.bfloat16), w3bd.astype(jnp.bfloat16),
            b1t, b2t, b3t)


@jax.jit
def _dqn_fused(x, w1, b1, w2, b2, w3, b3, w4, b4):
    batch = x.shape[0]

    w1bd, w2bd, w3bd, b1t, b2t, b3t = _pack_block_diag(w1, w2, w3, b1, b2, b3)
    w4b = w4.astype(jnp.bfloat16)
    b4t = b4[None, :]

    rows = _PACK * _TB
    grid = (pl.cdiv(batch, rows),)

    def chunk_spec(j):
        return pl.BlockSpec((_TB, x.shape[1]),
                            lambda i, j=j: (_PACK * i + j, 0))

    resident = lambda shape: pl.BlockSpec(shape, lambda i: (0,) * len(shape))
    return pl.pallas_call(
        _mlp_body,
        out_shape=jax.ShapeDtypeStruct((batch, 2), jnp.float32),
        grid=grid,
        in_specs=[
            chunk_spec(0), chunk_spec(1), chunk_spec(2), chunk_spec(3),
            resident(w1bd.shape),
            resident(w2bd.shape),
            resident(w3bd.shape),
            resident(w4b.shape),
            resident(b1t.shape),
            resident(b2t.shape),
            resident(b3t.shape),
            resident(b4t.shape),
        ],
        out_specs=pl.BlockSpec((rows, 2), lambda i: (i, 0)),
        compiler_params=pltpu.CompilerParams(
            dimension_semantics=("parallel",)),
    )(x, x, x, x, w1bd, w2bd, w3bd, w4b, b1t, b2t, b3t, b4t)


def kernel(x, w1, b1, w2, b2, w3, b3, w4, b4):
    return _dqn_fused(x, w1, b1, w2, b2, w3, b3, w4, b4)
